# baseline (speedup 1.0000x reference)
import functools

import jax
import jax.numpy as jnp
from jax import lax
from jax.experimental import pallas as pl
from jax.experimental.pallas import tpu as pltpu
from jax.experimental.pallas import tpu_sc as plsc

K = 2048
NB = 1024
CAND = 2560
CANDBUF = CAND + 16
UNROLL = 25


def _sc_select(logits):
    B, V = logits.shape
    n_iters = V // (16 * UNROLL)
    mesh = plsc.VectorSubcoreMesh(core_axis_name="c", subcore_axis_name="s")

    @functools.partial(
        pl.kernel,
        out_type=[
            jax.ShapeDtypeStruct((B, CAND), jnp.int32),
            jax.ShapeDtypeStruct((B, 16), jnp.int32),
            jax.ShapeDtypeStruct((B, 16), jnp.float32),
        ],
        mesh=mesh,
        compiler_params=pltpu.CompilerParams(needs_layout_passes=False),
        scratch_types=[
            pltpu.VMEM((V,), jnp.float32),
            pltpu.VMEM((NB * 16,), jnp.int32),
            pltpu.VMEM((NB,), jnp.int32),
            pltpu.VMEM((CANDBUF,), jnp.int32),
            pltpu.VMEM((16,), jnp.int32),
            pltpu.VMEM((16,), jnp.float32),
        ],
    )
    def sel(logits_hbm, cand_hbm, cnt_hbm, max_hbm,
            rowbuf, hist, totals, candbuf, cnt16, max16):
        n_cores = 2
        wid = lax.axis_index("s") * n_cores + lax.axis_index("c")
        rows_per_w = B // 32
        lanes = lax.iota(jnp.int32, 16)
        lane_base = lanes * NB

        for rr in range(rows_per_w):
            row = wid * rows_per_w + rr
            pltpu.sync_copy(logits_hbm.at[row], rowbuf)

            zeros16 = jnp.zeros((16,), jnp.int32)

            def zero_hist(i):
                hist[pl.ds(i * 16, 16)] = zeros16
            plsc.parallel_loop(0, NB, unroll=8)(zero_hist)

            def zero_cand(i):
                candbuf[pl.ds(i * 16, 16)] = zeros16
            plsc.parallel_loop(0, CANDBUF // 16, unroll=8)(zero_cand)

            ones = jnp.ones((16,), jnp.int32)

            def pass1(j, mx):
                x = rowbuf[pl.ds(j * 16, 16)]
                b = jnp.clip(((x + 2.0) * 64.0).astype(jnp.int32),
                             0, NB - 1)
                plsc.addupdate_scatter(hist, [lane_base + b], ones)
                return jnp.maximum(mx, x)
            mxv = plsc.parallel_loop(
                0, V // 16, unroll=UNROLL,
                carry=jnp.full((16,), -jnp.inf, jnp.float32))(pass1)
            rowmax = jnp.max(mxv)

            def reduce_hist(i):
                acc = hist[pl.ds(i * 16, 16)]
                for l in range(1, 16):
                    acc = acc + hist[pl.ds(l * NB + i * 16, 16)]
                totals[pl.ds(i * 16, 16)] = acc
            plsc.parallel_loop(0, NB // 16, unroll=2)(reduce_hist)

            def find_bin(i_, carry):
                b_star, acc = carry
                i = NB // 16 - 1 - i_
                t = totals[pl.ds(i * 16, 16)]
                tr = lax.rev(t, (0,))
                sfx = plsc.cumsum(tr) + acc
                m = sfx >= K
                hit = plsc.all_reduce_population_count(m)[0] > 0
                k0 = plsc.all_reduce_ffs(m)[0]
                cand_b = i * 16 + 15 - k0
                b_new = jnp.where((b_star < 0) & hit, cand_b, b_star)
                return b_new, acc + jnp.sum(t)
            b_star, _ = lax.fori_loop(0, NB // 16, find_bin,
                                      (jnp.int32(-1), jnp.int32(0)))
            b_star = jnp.maximum(b_star, 0)

            def pass2(j, wp):
                x = rowbuf[pl.ds(j * 16, 16)]
                b = jnp.clip(((x + 2.0) * 64.0).astype(jnp.int32),
                             0, NB - 1)
                m = b >= b_star
                safe = jnp.broadcast_to(wp + 16 <= CANDBUF, (16,))
                mm = m & safe
                idx = j * 16 + lanes
                plsc.store_compressed(candbuf.at[pl.ds(wp, 16)], idx,
                                      mask=mm)
                return wp + plsc.all_reduce_population_count(mm)[0]
            wp = plsc.parallel_loop(0, V // 16, unroll=UNROLL,
                                    carry=jnp.int32(0))(pass2)

            pltpu.sync_copy(candbuf.at[pl.ds(0, CAND)], cand_hbm.at[row])
            cnt16[...] = jnp.broadcast_to(wp, (16,))
            pltpu.sync_copy(cnt16, cnt_hbm.at[row])
            max16[...] = jnp.broadcast_to(rowmax, (16,))
            pltpu.sync_copy(max16, max_hbm.at[row])

    return sel(logits)


def kernel(logits, temperatures, top_ks, top_ps, min_ps, u):
    B, V = logits.shape
    top_ks = top_ks.astype(jnp.int32)
    T = temperatures[:, None]

    cand_idx, cnt, rowmax = _sc_select(logits)
    cnt = jnp.minimum(cnt[:, 0], CAND)

    mx = rowmax[:, :1] / temperatures[:, None]
    D = jnp.sum(jnp.exp(logits / T - mx), axis=-1, keepdims=True)
    lg_cand = jnp.take_along_axis(logits, cand_idx, axis=-1)
    p_cand = jnp.exp(lg_cand / T - mx) / D

    valid = jnp.arange(CAND, dtype=jnp.int32)[None, :] < cnt[:, None]
    negp = jnp.where(valid, -p_cand, 1.0)
    idxs = jnp.where(valid, cand_idx, V)
    negp_s, idx_s = jax.lax.sort((negp, idxs), dimension=-1, num_keys=2)
    psort = -negp_s[:, :K]
    order = idx_s[:, :K]

    probs_sum = jnp.cumsum(psort, axis=-1)

    ranks = jnp.arange(K, dtype=jnp.int32)[None, :]
    ps = jnp.where(ranks >= top_ks[:, None], 0.0, psort)
    mask_p = probs_sum - psort > top_ps[:, None]
    ps = jnp.where(mask_p, 0.0, ps)
    thr = ps[:, 0] * min_ps
    ps = jnp.where(ps < thr[:, None], 0.0, ps)

    denom = jnp.sum(ps, axis=-1, keepdims=True)
    cdf = jnp.cumsum(ps, axis=-1) / denom
    sampled = jnp.clip(jnp.sum((cdf < u[:, None]).astype(jnp.int32), axis=-1),
                       0, V - 1)
    samp_c = jnp.minimum(sampled, K - 1)
    next_ids = jnp.take_along_axis(order, samp_c[:, None], axis=-1)[:, 0]
    next_ids = next_ids.astype(jnp.int32)

    crossed_in_block = probs_sum[:, K - 1] - psort[:, K - 1] > top_ps
    S2_exact = jnp.sum(jnp.where(mask_p, 0.0, psort), axis=-1)
    S2_approx = top_ps + 0.5 * psort[:, K - 1]
    S2 = jnp.where(crossed_in_block, S2_exact, S2_approx)

    rows = jnp.arange(B)
    gathered = psort[rows, samp_c] / S2
    next_logprobs = jnp.log(gathered)

    top_vals = jnp.log(psort[:, :5] / S2[:, None])
    top_idx = order[:, :5].astype(jnp.int32)

    return next_ids, next_logprobs, top_vals, top_idx


# speedup vs baseline: 1.1071x; 1.1071x over previous
import functools

import jax
import jax.numpy as jnp
from jax import lax
from jax.experimental import pallas as pl
from jax.experimental.pallas import tpu as pltpu
from jax.experimental.pallas import tpu_sc as plsc

K = 2048
NB = 1024
CAND = 2560
CANDBUF = CAND + 16
UNROLL = 10


def _sc_select(logits):
    B, V = logits.shape
    n_iters = V // (16 * UNROLL)
    mesh = plsc.VectorSubcoreMesh(core_axis_name="c", subcore_axis_name="s")

    @functools.partial(
        pl.kernel,
        out_type=[
            jax.ShapeDtypeStruct((B, CAND), jnp.int32),
            jax.ShapeDtypeStruct((B, 16), jnp.int32),
            jax.ShapeDtypeStruct((B, 16), jnp.float32),
        ],
        mesh=mesh,
        compiler_params=pltpu.CompilerParams(needs_layout_passes=False),
        scratch_types=[
            pltpu.VMEM((V,), jnp.float32),
            pltpu.VMEM((NB * 16,), jnp.int32),
            pltpu.VMEM((NB,), jnp.int32),
            pltpu.VMEM((CANDBUF,), jnp.int32),
            pltpu.VMEM((16,), jnp.int32),
            pltpu.VMEM((16,), jnp.float32),
        ],
    )
    def sel(logits_hbm, cand_hbm, cnt_hbm, max_hbm,
            rowbuf, hist, totals, candbuf, cnt16, max16):
        n_cores = 2
        wid = lax.axis_index("s") * n_cores + lax.axis_index("c")
        rows_per_w = B // 32
        lanes = lax.iota(jnp.int32, 16)
        lane_base = lanes * NB

        for rr in range(rows_per_w):
            row = wid * rows_per_w + rr
            pltpu.sync_copy(logits_hbm.at[row], rowbuf)

            zeros16 = jnp.zeros((16,), jnp.int32)

            def zero_hist(i):
                hist[pl.ds(i * 16, 16)] = zeros16
            plsc.parallel_loop(0, NB, unroll=8)(zero_hist)

            def zero_cand(i):
                candbuf[pl.ds(i * 16, 16)] = zeros16
            plsc.parallel_loop(0, CANDBUF // 16, unroll=8)(zero_cand)

            ones = jnp.ones((16,), jnp.int32)

            def pass1(j, mx):
                x = rowbuf[pl.ds(j * 16, 16)]
                b = jnp.clip(((x + 2.0) * 64.0).astype(jnp.int32),
                             0, NB - 1)
                plsc.addupdate_scatter(hist, [lane_base + b], ones)
                return jnp.maximum(mx, x)
            mxv = plsc.parallel_loop(
                0, V // 16, unroll=UNROLL,
                carry=jnp.full((16,), -jnp.inf, jnp.float32))(pass1)
            rowmax = jnp.max(mxv)

            def reduce_hist(i):
                acc = hist[pl.ds(i * 16, 16)]
                for l in range(1, 16):
                    acc = acc + hist[pl.ds(l * NB + i * 16, 16)]
                totals[pl.ds(i * 16, 16)] = acc
            plsc.parallel_loop(0, NB // 16, unroll=2)(reduce_hist)

            def find_bin(i_, carry):
                b_star, acc = carry
                i = NB // 16 - 1 - i_
                t = totals[pl.ds(i * 16, 16)]
                tr = lax.rev(t, (0,))
                sfx = plsc.cumsum(tr) + acc
                m = sfx >= K
                hit = plsc.all_reduce_population_count(m)[0] > 0
                k0 = plsc.all_reduce_ffs(m)[0]
                cand_b = i * 16 + 15 - k0
                b_new = jnp.where((b_star < 0) & hit, cand_b, b_star)
                return b_new, acc + jnp.sum(t)
            b_star, _ = lax.fori_loop(0, NB // 16, find_bin,
                                      (jnp.int32(-1), jnp.int32(0)))
            b_star = jnp.maximum(b_star, 0)

            def pass2(j, wp):
                x = rowbuf[pl.ds(j * 16, 16)]
                b = jnp.clip(((x + 2.0) * 64.0).astype(jnp.int32),
                             0, NB - 1)
                m = b >= b_star
                safe = jnp.broadcast_to(wp + 16 <= CANDBUF, (16,))
                mm = m & safe
                idx = j * 16 + lanes
                plsc.store_compressed(candbuf.at[pl.ds(wp, 16)], idx,
                                      mask=mm)
                return wp + plsc.all_reduce_population_count(mm)[0]
            wp = plsc.parallel_loop(0, V // 16, unroll=UNROLL,
                                    carry=jnp.int32(0))(pass2)

            pltpu.sync_copy(candbuf.at[pl.ds(0, CAND)], cand_hbm.at[row])
            cnt16[...] = jnp.broadcast_to(wp, (16,))
            pltpu.sync_copy(cnt16, cnt_hbm.at[row])
            max16[...] = jnp.broadcast_to(rowmax, (16,))
            pltpu.sync_copy(max16, max_hbm.at[row])

    return sel(logits)


def kernel(logits, temperatures, top_ks, top_ps, min_ps, u):
    B, V = logits.shape
    top_ks = top_ks.astype(jnp.int32)
    T = temperatures[:, None]

    cand_idx, cnt, rowmax = _sc_select(logits)
    cnt = jnp.minimum(cnt[:, 0], CAND)

    mx = rowmax[:, :1] / temperatures[:, None]
    D = jnp.sum(jnp.exp(logits / T - mx), axis=-1, keepdims=True)
    lg_cand = jnp.take_along_axis(logits, cand_idx, axis=-1)
    p_cand = jnp.exp(lg_cand / T - mx) / D

    valid = jnp.arange(CAND, dtype=jnp.int32)[None, :] < cnt[:, None]
    p_masked = jnp.where(valid, p_cand, -1.0)
    psort, pos = jax.lax.top_k(p_masked, K)
    order = jnp.take_along_axis(cand_idx, pos, axis=-1)

    probs_sum = jnp.cumsum(psort, axis=-1)

    ranks = jnp.arange(K, dtype=jnp.int32)[None, :]
    ps = jnp.where(ranks >= top_ks[:, None], 0.0, psort)
    mask_p = probs_sum - psort > top_ps[:, None]
    ps = jnp.where(mask_p, 0.0, ps)
    thr = ps[:, 0] * min_ps
    ps = jnp.where(ps < thr[:, None], 0.0, ps)

    denom = jnp.sum(ps, axis=-1, keepdims=True)
    cdf = jnp.cumsum(ps, axis=-1) / denom
    sampled = jnp.clip(jnp.sum((cdf < u[:, None]).astype(jnp.int32), axis=-1),
                       0, V - 1)
    samp_c = jnp.minimum(sampled, K - 1)
    next_ids = jnp.take_along_axis(order, samp_c[:, None], axis=-1)[:, 0]
    next_ids = next_ids.astype(jnp.int32)

    crossed_in_block = probs_sum[:, K - 1] - psort[:, K - 1] > top_ps
    S2_exact = jnp.sum(jnp.where(mask_p, 0.0, psort), axis=-1)
    S2_approx = top_ps + 0.5 * psort[:, K - 1]
    S2 = jnp.where(crossed_in_block, S2_exact, S2_approx)

    rows = jnp.arange(B)
    gathered = psort[rows, samp_c] / S2
    next_logprobs = jnp.log(gathered)

    top_vals = jnp.log(psort[:, :5] / S2[:, None])
    top_idx = order[:, :5].astype(jnp.int32)

    return next_ids, next_logprobs, top_vals, top_idx


# speedup vs baseline: 1.1750x; 1.0614x over previous
import functools

import jax
import jax.numpy as jnp
from jax import lax
from jax.experimental import pallas as pl
from jax.experimental.pallas import tpu as pltpu
from jax.experimental.pallas import tpu_sc as plsc

K = 2048
NB = 1024
CAND = 2304
CANDBUF = CAND + 16
UNROLL = 10


def _sc_select(logits):
    B, V = logits.shape
    n_iters = V // (16 * UNROLL)
    mesh = plsc.VectorSubcoreMesh(core_axis_name="c", subcore_axis_name="s")

    @functools.partial(
        pl.kernel,
        out_type=[
            jax.ShapeDtypeStruct((B, CAND), jnp.int32),
            jax.ShapeDtypeStruct((B, CAND), jnp.float32),
            jax.ShapeDtypeStruct((B, 16), jnp.int32),
            jax.ShapeDtypeStruct((B, 16), jnp.float32),
        ],
        mesh=mesh,
        compiler_params=pltpu.CompilerParams(needs_layout_passes=False),
        scratch_types=[
            pltpu.VMEM((V,), jnp.float32),
            pltpu.VMEM((NB * 16,), jnp.int32),
            pltpu.VMEM((NB,), jnp.int32),
            pltpu.VMEM((CANDBUF,), jnp.int32),
            pltpu.VMEM((CANDBUF,), jnp.float32),
            pltpu.VMEM((16,), jnp.int32),
            pltpu.VMEM((16,), jnp.float32),
        ],
    )
    def sel(logits_hbm, cand_hbm, val_hbm, cnt_hbm, max_hbm,
            rowbuf, hist, totals, candbuf, valbuf, cnt16, max16):
        n_cores = 2
        wid = lax.axis_index("s") * n_cores + lax.axis_index("c")
        rows_per_w = B // 32
        lanes = lax.iota(jnp.int32, 16)
        lane_base = lanes * NB

        for rr in range(rows_per_w):
            row = wid * rows_per_w + rr
            pltpu.sync_copy(logits_hbm.at[row], rowbuf)

            zeros16 = jnp.zeros((16,), jnp.int32)

            def zero_hist(i):
                hist[pl.ds(i * 16, 16)] = zeros16
            plsc.parallel_loop(0, NB, unroll=8)(zero_hist)

            def zero_cand(i):
                candbuf[pl.ds(i * 16, 16)] = zeros16
                valbuf[pl.ds(i * 16, 16)] = jnp.zeros((16,), jnp.float32)
            plsc.parallel_loop(0, CANDBUF // 16, unroll=8)(zero_cand)

            ones = jnp.ones((16,), jnp.int32)

            def pass1(j, mx):
                x = rowbuf[pl.ds(j * 16, 16)]
                b = jnp.clip(((x + 2.0) * 64.0).astype(jnp.int32),
                             0, NB - 1)
                plsc.addupdate_scatter(hist, [lane_base + b], ones)
                return jnp.maximum(mx, x)
            mxv = plsc.parallel_loop(
                0, V // 16, unroll=UNROLL,
                carry=jnp.full((16,), -jnp.inf, jnp.float32))(pass1)
            rowmax = jnp.max(mxv)

            def reduce_hist(i):
                acc = hist[pl.ds(i * 16, 16)]
                for l in range(1, 16):
                    acc = acc + hist[pl.ds(l * NB + i * 16, 16)]
                totals[pl.ds(i * 16, 16)] = acc
            plsc.parallel_loop(0, NB // 16, unroll=2)(reduce_hist)

            def find_bin(i_, carry):
                b_star, acc = carry
                i = NB // 16 - 1 - i_
                t = totals[pl.ds(i * 16, 16)]
                tr = lax.rev(t, (0,))
                sfx = plsc.cumsum(tr) + acc
                m = sfx >= K
                hit = plsc.all_reduce_population_count(m)[0] > 0
                k0 = plsc.all_reduce_ffs(m)[0]
                cand_b = i * 16 + 15 - k0
                b_new = jnp.where((b_star < 0) & hit, cand_b, b_star)
                return b_new, acc + jnp.sum(t)
            b_star, _ = lax.fori_loop(0, NB // 16, find_bin,
                                      (jnp.int32(-1), jnp.int32(0)))
            b_star = jnp.maximum(b_star, 0)

            def pass2(j, wp):
                x = rowbuf[pl.ds(j * 16, 16)]
                b = jnp.clip(((x + 2.0) * 64.0).astype(jnp.int32),
                             0, NB - 1)
                m = b >= b_star
                safe = jnp.broadcast_to(wp + 16 <= CANDBUF, (16,))
                mm = m & safe
                idx = j * 16 + lanes
                plsc.store_compressed(candbuf.at[pl.ds(wp, 16)], idx,
                                      mask=mm)
                plsc.store_compressed(valbuf.at[pl.ds(wp, 16)], x, mask=mm)
                return wp + plsc.all_reduce_population_count(mm)[0]
            wp = plsc.parallel_loop(0, V // 16, unroll=UNROLL,
                                    carry=jnp.int32(0))(pass2)

            pltpu.sync_copy(candbuf.at[pl.ds(0, CAND)], cand_hbm.at[row])
            pltpu.sync_copy(valbuf.at[pl.ds(0, CAND)], val_hbm.at[row])
            cnt16[...] = jnp.broadcast_to(wp, (16,))
            pltpu.sync_copy(cnt16, cnt_hbm.at[row])
            max16[...] = jnp.broadcast_to(rowmax, (16,))
            pltpu.sync_copy(max16, max_hbm.at[row])

    return sel(logits)


def kernel(logits, temperatures, top_ks, top_ps, min_ps, u):
    B, V = logits.shape
    top_ks = top_ks.astype(jnp.int32)
    T = temperatures[:, None]

    cand_idx, lg_cand, cnt, rowmax = _sc_select(logits)
    cnt = jnp.minimum(cnt[:, 0], CAND)

    mx = rowmax[:, :1] / temperatures[:, None]
    D = jnp.sum(jnp.exp(logits / T - mx), axis=-1, keepdims=True)
    p_cand = jnp.exp(lg_cand / T - mx) / D

    valid = jnp.arange(CAND, dtype=jnp.int32)[None, :] < cnt[:, None]
    p_masked = jnp.where(valid, p_cand, -1.0)
    psort, pos = jax.lax.top_k(p_masked, K)
    order = jnp.take_along_axis(cand_idx, pos, axis=-1)

    probs_sum = jnp.cumsum(psort, axis=-1)

    ranks = jnp.arange(K, dtype=jnp.int32)[None, :]
    ps = jnp.where(ranks >= top_ks[:, None], 0.0, psort)
    mask_p = probs_sum - psort > top_ps[:, None]
    ps = jnp.where(mask_p, 0.0, ps)
    thr = ps[:, 0] * min_ps
    ps = jnp.where(ps < thr[:, None], 0.0, ps)

    denom = jnp.sum(ps, axis=-1, keepdims=True)
    cdf = jnp.cumsum(ps, axis=-1) / denom
    sampled = jnp.clip(jnp.sum((cdf < u[:, None]).astype(jnp.int32), axis=-1),
                       0, V - 1)
    samp_c = jnp.minimum(sampled, K - 1)
    next_ids = jnp.take_along_axis(order, samp_c[:, None], axis=-1)[:, 0]
    next_ids = next_ids.astype(jnp.int32)

    crossed_in_block = probs_sum[:, K - 1] - psort[:, K - 1] > top_ps
    S2_exact = jnp.sum(jnp.where(mask_p, 0.0, psort), axis=-1)
    S2_approx = top_ps + 0.5 * psort[:, K - 1]
    S2 = jnp.where(crossed_in_block, S2_exact, S2_approx)

    rows = jnp.arange(B)
    gathered = psort[rows, samp_c] / S2
    next_logprobs = jnp.log(gathered)

    top_vals = jnp.log(psort[:, :5] / S2[:, None])
    top_idx = order[:, :5].astype(jnp.int32)

    return next_ids, next_logprobs, top_vals, top_idx
